# Initial kernel scaffold; baseline (speedup 1.0000x reference)
import functools

import jax
import jax.numpy as jnp
from jax import lax
from jax.experimental import pallas as pl
from jax.experimental.pallas import tpu as pltpu
import jax.experimental.pallas.tpu_sc as plsc

_NC = 2
_NS = 16
_CH = 128
_BR = 1000


def _edge_deg(dst2d, ones_h, zrows):
    n_rows = dst2d.shape[0]
    cpw = n_rows // (_NC * _NS)
    rpt = zrows.shape[0]
    n_acc = _NS * rpt
    mesh = plsc.VectorSubcoreMesh(core_axis_name="c", subcore_axis_name="s")

    @functools.partial(
        pl.kernel,
        out_type=jax.ShapeDtypeStruct((_NC, n_acc, 1), jnp.float32),
        mesh=mesh,
        scratch_types=[
            pltpu.VMEM((cpw, _CH), jnp.int32),
            pltpu.VMEM((_CH, 1), jnp.float32),
            pltpu.VMEM_SHARED((n_acc, 1), jnp.float32),
        ],
    )
    def k(dst_hbm, ones_hbm, z_hbm, out_hbm, dst_v, ones_v, acc):
        cid = lax.axis_index("c")
        sid = lax.axis_index("s")
        wid = sid * _NC + cid
        pltpu.sync_copy(dst_hbm.at[pl.ds(wid * cpw, cpw)], dst_v)
        pltpu.sync_copy(ones_hbm, ones_v)
        pltpu.sync_copy(z_hbm, acc.at[pl.ds(sid * rpt, rpt)])
        plsc.subcore_barrier()

        def body(j, carry):
            pltpu.sync_copy(ones_v, acc.at[dst_v.at[j]], add=True)
            return carry

        lax.fori_loop(0, cpw, body, 0)
        plsc.subcore_barrier()
        pltpu.sync_copy(acc.at[pl.ds(sid * rpt, rpt)],
                        out_hbm.at[cid, pl.ds(sid * rpt, rpt)])

    return k(dst2d, ones_h, zrows)


def _edge_segsum(g, src2d, dst2d, zrows):
    D = g.shape[1]
    n_rows = src2d.shape[0]
    cpw = n_rows // (_NC * _NS)
    rpt = zrows.shape[0]
    n_acc = _NS * rpt
    mesh = plsc.VectorSubcoreMesh(core_axis_name="c", subcore_axis_name="s")

    @functools.partial(
        pl.kernel,
        out_type=jax.ShapeDtypeStruct((_NC, n_acc, D), jnp.float32),
        mesh=mesh,
        scratch_types=[
            pltpu.VMEM((cpw, _CH), jnp.int32),
            pltpu.VMEM((cpw, _CH), jnp.int32),
            pltpu.VMEM((_CH, D), jnp.float32),
            pltpu.VMEM_SHARED((n_acc, D), jnp.float32),
            pltpu.SemaphoreType.DMA,
        ],
    )
    def k(g_hbm, src_hbm, dst_hbm, z_hbm, out_hbm,
          src_v, dst_v, rows_v, acc, sem):
        cid = lax.axis_index("c")
        sid = lax.axis_index("s")
        wid = sid * _NC + cid
        pltpu.sync_copy(src_hbm.at[pl.ds(wid * cpw, cpw)], src_v)
        pltpu.sync_copy(dst_hbm.at[pl.ds(wid * cpw, cpw)], dst_v)
        pltpu.sync_copy(z_hbm, acc.at[pl.ds(sid * rpt, rpt)])
        plsc.subcore_barrier()

        def body(j, carry):
            pltpu.async_copy(g_hbm.at[src_v.at[j]], rows_v, sem).wait()
            pltpu.sync_copy(rows_v, acc.at[dst_v.at[j]], add=True)
            return carry

        lax.fori_loop(0, cpw, body, 0)
        plsc.subcore_barrier()
        pltpu.sync_copy(acc.at[pl.ds(sid * rpt, rpt)],
                        out_hbm.at[cid, pl.ds(sid * rpt, rpt)])

    return k(g, src2d, dst2d, zrows)


def _tc_matmul(x, W):
    N, F = x.shape
    H = W.shape[1]

    def body(x_ref, w_ref, o_ref):
        o_ref[...] = jnp.dot(x_ref[...], w_ref[...],
                             preferred_element_type=jnp.float32)

    return pl.pallas_call(
        body,
        grid=(N // _BR,),
        in_specs=[pl.BlockSpec((_BR, F), lambda i: (i, 0)),
                  pl.BlockSpec((F, H), lambda i: (0, 0))],
        out_specs=pl.BlockSpec((_BR, H), lambda i: (i, 0)),
        out_shape=jax.ShapeDtypeStruct((N, H), jnp.float32),
    )(x, W)


def _tc_scale(h, degP):
    N, H = h.shape

    def body(h_ref, d0_ref, d1_ref, g_ref, dinv_ref):
        deg = d0_ref[0] + d1_ref[0] + 1.0
        dinv = lax.rsqrt(deg)
        dinv_ref[...] = dinv
        g_ref[...] = h_ref[...] * dinv

    return pl.pallas_call(
        body,
        grid=(N // _BR,),
        in_specs=[pl.BlockSpec((_BR, H), lambda i: (i, 0)),
                  pl.BlockSpec((1, _BR, 1), lambda i: (0, i, 0)),
                  pl.BlockSpec((1, _BR, 1), lambda i: (1, i, 0))],
        out_specs=[pl.BlockSpec((_BR, H), lambda i: (i, 0)),
                   pl.BlockSpec((_BR, 1), lambda i: (i, 0))],
        out_shape=[jax.ShapeDtypeStruct((N, H), jnp.float32),
                   jax.ShapeDtypeStruct((N, 1), jnp.float32)],
    )(h, degP, degP)


def _tc_mid(g1, sp1, dinv, b1, W2):
    N, H = g1.shape
    C = W2.shape[1]

    def body(g1_ref, s0_ref, s1_ref, dinv_ref, b1_ref, w2_ref, g2_ref):
        s = s0_ref[0] + s1_ref[0] + g1_ref[...]
        t = jnp.maximum(s * dinv_ref[...] + b1_ref[...], 0.0)
        g2_ref[...] = jnp.dot(t, w2_ref[...],
                              preferred_element_type=jnp.float32) * dinv_ref[...]

    return pl.pallas_call(
        body,
        grid=(N // _BR,),
        in_specs=[pl.BlockSpec((_BR, H), lambda i: (i, 0)),
                  pl.BlockSpec((1, _BR, H), lambda i: (0, i, 0)),
                  pl.BlockSpec((1, _BR, H), lambda i: (1, i, 0)),
                  pl.BlockSpec((_BR, 1), lambda i: (i, 0)),
                  pl.BlockSpec((1, H), lambda i: (0, 0)),
                  pl.BlockSpec((H, C), lambda i: (0, 0))],
        out_specs=pl.BlockSpec((_BR, C), lambda i: (i, 0)),
        out_shape=jax.ShapeDtypeStruct((N, C), jnp.float32),
    )(g1, sp1, sp1, dinv, b1, W2)


def _tc_out(g2, sp2, dinv, b2):
    N, C = g2.shape

    def body(g2_ref, s0_ref, s1_ref, dinv_ref, b2_ref, o_ref):
        s = s0_ref[0] + s1_ref[0] + g2_ref[...]
        o_ref[...] = jax.nn.sigmoid(s * dinv_ref[...] + b2_ref[...])

    return pl.pallas_call(
        body,
        grid=(N // _BR,),
        in_specs=[pl.BlockSpec((_BR, C), lambda i: (i, 0)),
                  pl.BlockSpec((1, _BR, C), lambda i: (0, i, 0)),
                  pl.BlockSpec((1, _BR, C), lambda i: (1, i, 0)),
                  pl.BlockSpec((_BR, 1), lambda i: (i, 0)),
                  pl.BlockSpec((1, C), lambda i: (0, 0))],
        out_specs=pl.BlockSpec((_BR, C), lambda i: (i, 0)),
        out_shape=jax.ShapeDtypeStruct((N, C), jnp.float32),
    )(g2, sp2, sp2, dinv, b2)


def kernel(x, edge_index, W1, b1, W2, b2):
    N, F = x.shape
    H = W1.shape[1]
    C = W2.shape[1]
    E = edge_index.shape[1]

    src = edge_index[0].astype(jnp.int32)
    dst = edge_index[1].astype(jnp.int32)

    NW = _NC * _NS
    cpw = -(-E // (_CH * NW))
    cpw += cpw % 2
    Epad = NW * cpw * _CH
    src2 = jnp.concatenate(
        [src, jnp.zeros((Epad - E,), jnp.int32)]).reshape(NW * cpw, _CH)
    dst2 = jnp.concatenate(
        [dst, jnp.full((Epad - E,), N, jnp.int32)]).reshape(NW * cpw, _CH)

    rpt = -(-(N + 1) // _NS)
    rpt = -(-rpt // 8) * 8
    zH = jnp.zeros((rpt, H), jnp.float32)
    zC = jnp.zeros((rpt, C), jnp.float32)
    z1 = jnp.zeros((rpt, 1), jnp.float32)
    ones = jnp.ones((_CH, 1), jnp.float32)

    h1 = _tc_matmul(x, W1)
    degP = _edge_deg(dst2, ones, z1)
    g1, dinv = _tc_scale(h1, degP)
    sp1 = _edge_segsum(g1, src2, dst2, zH)
    g2 = _tc_mid(g1, sp1, dinv, b1.reshape(1, H), W2)
    sp2 = _edge_segsum(g2, src2, dst2, zC)
    return _tc_out(g2, sp2, dinv, b2.reshape(1, C))


# speedup vs baseline: 18.5725x; 18.5725x over previous
import functools

import jax
import jax.numpy as jnp
from jax import lax
from jax.experimental import pallas as pl
from jax.experimental.pallas import tpu as pltpu
import jax.experimental.pallas.tpu_sc as plsc

_NC = 2
_NS = 16
_CH = 128
_BR = 1000


def _edge_deg(dst2d, ones_h, zrows):
    n_rows = dst2d.shape[0]
    cpw = n_rows // (_NC * _NS)
    rpt = zrows.shape[0]
    n_acc = _NS * rpt
    mesh = plsc.VectorSubcoreMesh(core_axis_name="c", subcore_axis_name="s")

    @functools.partial(
        pl.kernel,
        out_type=jax.ShapeDtypeStruct((_NC, n_acc, 1), jnp.float32),
        mesh=mesh,
        scratch_types=[
            pltpu.VMEM((cpw, _CH), jnp.int32),
            pltpu.VMEM((_CH, 1), jnp.float32),
            pltpu.VMEM_SHARED((n_acc, 1), jnp.float32),
        ],
        compiler_params=pltpu.CompilerParams(use_tc_tiling_on_sc=False),
    )
    def k(dst_hbm, ones_hbm, z_hbm, out_hbm, dst_v, ones_v, acc):
        cid = lax.axis_index("c")
        sid = lax.axis_index("s")
        wid = sid * _NC + cid
        pltpu.sync_copy(dst_hbm.at[pl.ds(wid * cpw, cpw)], dst_v)
        pltpu.sync_copy(ones_hbm, ones_v)
        pltpu.sync_copy(z_hbm, acc.at[pl.ds(sid * rpt, rpt)])
        plsc.subcore_barrier()

        def body(j, carry):
            pltpu.sync_copy(ones_v, acc.at[dst_v.at[j]], add=True)
            return carry

        lax.fori_loop(0, cpw, body, 0)
        plsc.subcore_barrier()
        pltpu.sync_copy(acc.at[pl.ds(sid * rpt, rpt)],
                        out_hbm.at[cid, pl.ds(sid * rpt, rpt)])

    return k(dst2d, ones_h, zrows)


def _edge_segsum(g, src2d, dst2d, zrows):
    D = g.shape[1]
    n_rows = src2d.shape[0]
    cpw = n_rows // (_NC * _NS)
    rpt = zrows.shape[0]
    n_acc = _NS * rpt
    mesh = plsc.VectorSubcoreMesh(core_axis_name="c", subcore_axis_name="s")

    @functools.partial(
        pl.kernel,
        out_type=jax.ShapeDtypeStruct((_NC, n_acc, D), jnp.float32),
        mesh=mesh,
        scratch_types=[
            pltpu.VMEM((cpw, _CH), jnp.int32),
            pltpu.VMEM((cpw, _CH), jnp.int32),
            pltpu.VMEM((_CH, D), jnp.float32),
            pltpu.VMEM_SHARED((n_acc, D), jnp.float32),
            pltpu.SemaphoreType.DMA,
        ],
        compiler_params=pltpu.CompilerParams(use_tc_tiling_on_sc=False),
    )
    def k(g_hbm, src_hbm, dst_hbm, z_hbm, out_hbm,
          src_v, dst_v, rows_v, acc, sem):
        cid = lax.axis_index("c")
        sid = lax.axis_index("s")
        wid = sid * _NC + cid
        pltpu.sync_copy(src_hbm.at[pl.ds(wid * cpw, cpw)], src_v)
        pltpu.sync_copy(dst_hbm.at[pl.ds(wid * cpw, cpw)], dst_v)
        pltpu.sync_copy(z_hbm, acc.at[pl.ds(sid * rpt, rpt)])
        plsc.subcore_barrier()

        def body(j, carry):
            pltpu.async_copy(g_hbm.at[src_v.at[j]], rows_v, sem).wait()
            pltpu.sync_copy(rows_v, acc.at[dst_v.at[j]], add=True)
            return carry

        lax.fori_loop(0, cpw, body, 0)
        plsc.subcore_barrier()
        pltpu.sync_copy(acc.at[pl.ds(sid * rpt, rpt)],
                        out_hbm.at[cid, pl.ds(sid * rpt, rpt)])

    return k(g, src2d, dst2d, zrows)


def _tc_matmul(x, W):
    N, F = x.shape
    H = W.shape[1]

    def body(x_ref, w_ref, o_ref):
        o_ref[...] = jnp.dot(x_ref[...], w_ref[...],
                             preferred_element_type=jnp.float32)

    return pl.pallas_call(
        body,
        grid=(N // _BR,),
        in_specs=[pl.BlockSpec((_BR, F), lambda i: (i, 0)),
                  pl.BlockSpec((F, H), lambda i: (0, 0))],
        out_specs=pl.BlockSpec((_BR, H), lambda i: (i, 0)),
        out_shape=jax.ShapeDtypeStruct((N, H), jnp.float32),
    )(x, W)


def _tc_scale(h, degP):
    N, H = h.shape

    def body(h_ref, d0_ref, d1_ref, g_ref, dinv_ref):
        deg = d0_ref[0] + d1_ref[0] + 1.0
        dinv = lax.rsqrt(deg)
        dinv_ref[...] = dinv
        g_ref[...] = h_ref[...] * dinv

    return pl.pallas_call(
        body,
        grid=(N // _BR,),
        in_specs=[pl.BlockSpec((_BR, H), lambda i: (i, 0)),
                  pl.BlockSpec((1, _BR, 1), lambda i: (0, i, 0)),
                  pl.BlockSpec((1, _BR, 1), lambda i: (1, i, 0))],
        out_specs=[pl.BlockSpec((_BR, H), lambda i: (i, 0)),
                   pl.BlockSpec((_BR, 1), lambda i: (i, 0))],
        out_shape=[jax.ShapeDtypeStruct((N, H), jnp.float32),
                   jax.ShapeDtypeStruct((N, 1), jnp.float32)],
    )(h, degP, degP)


def _tc_mid(g1, sp1, dinv, b1, W2):
    N, H = g1.shape
    C = W2.shape[1]

    def body(g1_ref, s0_ref, s1_ref, dinv_ref, b1_ref, w2_ref, g2_ref):
        s = s0_ref[0] + s1_ref[0] + g1_ref[...]
        t = jnp.maximum(s * dinv_ref[...] + b1_ref[...], 0.0)
        g2_ref[...] = jnp.dot(t, w2_ref[...],
                              preferred_element_type=jnp.float32) * dinv_ref[...]

    return pl.pallas_call(
        body,
        grid=(N // _BR,),
        in_specs=[pl.BlockSpec((_BR, H), lambda i: (i, 0)),
                  pl.BlockSpec((1, _BR, H), lambda i: (0, i, 0)),
                  pl.BlockSpec((1, _BR, H), lambda i: (1, i, 0)),
                  pl.BlockSpec((_BR, 1), lambda i: (i, 0)),
                  pl.BlockSpec((1, H), lambda i: (0, 0)),
                  pl.BlockSpec((H, C), lambda i: (0, 0))],
        out_specs=pl.BlockSpec((_BR, C), lambda i: (i, 0)),
        out_shape=jax.ShapeDtypeStruct((N, C), jnp.float32),
    )(g1, sp1, sp1, dinv, b1, W2)


def _tc_out(g2, sp2, dinv, b2):
    N, C = g2.shape

    def body(g2_ref, s0_ref, s1_ref, dinv_ref, b2_ref, o_ref):
        s = s0_ref[0] + s1_ref[0] + g2_ref[...]
        o_ref[...] = jax.nn.sigmoid(s * dinv_ref[...] + b2_ref[...])

    return pl.pallas_call(
        body,
        grid=(N // _BR,),
        in_specs=[pl.BlockSpec((_BR, C), lambda i: (i, 0)),
                  pl.BlockSpec((1, _BR, C), lambda i: (0, i, 0)),
                  pl.BlockSpec((1, _BR, C), lambda i: (1, i, 0)),
                  pl.BlockSpec((_BR, 1), lambda i: (i, 0)),
                  pl.BlockSpec((1, C), lambda i: (0, 0))],
        out_specs=pl.BlockSpec((_BR, C), lambda i: (i, 0)),
        out_shape=jax.ShapeDtypeStruct((N, C), jnp.float32),
    )(g2, sp2, sp2, dinv, b2)


def kernel(x, edge_index, W1, b1, W2, b2):
    N, F = x.shape
    H = W1.shape[1]
    C = W2.shape[1]
    E = edge_index.shape[1]

    src = edge_index[0].astype(jnp.int32)
    dst = edge_index[1].astype(jnp.int32)

    NW = _NC * _NS
    cpw = -(-E // (_CH * NW))
    cpw += cpw % 2
    Epad = NW * cpw * _CH
    src2 = jnp.concatenate(
        [src, jnp.zeros((Epad - E,), jnp.int32)]).reshape(NW * cpw, _CH)
    dst2 = jnp.concatenate(
        [dst, jnp.full((Epad - E,), N, jnp.int32)]).reshape(NW * cpw, _CH)

    rpt = -(-(N + 1) // _NS)
    rpt = -(-rpt // 8) * 8
    zH = jnp.zeros((rpt, H), jnp.float32)
    zC = jnp.zeros((rpt, C), jnp.float32)
    z1 = jnp.zeros((rpt, 1), jnp.float32)
    ones = jnp.ones((_CH, 1), jnp.float32)

    h1 = _tc_matmul(x, W1)
    degP = _edge_deg(dst2, ones, z1)
    g1, dinv = _tc_scale(h1, degP)
    sp1 = _edge_segsum(g1, src2, dst2, zH)
    g2 = _tc_mid(g1, sp1, dinv, b1.reshape(1, H), W2)
    sp2 = _edge_segsum(g2, src2, dst2, zC)
    return _tc_out(g2, sp2, dinv, b2.reshape(1, C))


# speedup vs baseline: 21.8074x; 1.1742x over previous
import functools

import jax
import jax.numpy as jnp
from jax import lax
from jax.experimental import pallas as pl
from jax.experimental.pallas import tpu as pltpu
import jax.experimental.pallas.tpu_sc as plsc

_NC = 2
_NS = 16
_CH = 128
_BR = 1000


def _edge_deg(dst2d, ones_h, zrows):
    n_rows = dst2d.shape[0]
    cpw = n_rows // (_NC * _NS)
    rpt = zrows.shape[0]
    n_acc = _NS * rpt
    mesh = plsc.VectorSubcoreMesh(core_axis_name="c", subcore_axis_name="s")

    @functools.partial(
        pl.kernel,
        out_type=jax.ShapeDtypeStruct((_NC, n_acc, 1), jnp.float32),
        mesh=mesh,
        scratch_types=[
            pltpu.VMEM((cpw, _CH), jnp.int32),
            pltpu.VMEM((_CH, 1), jnp.float32),
            pltpu.VMEM_SHARED((n_acc, 1), jnp.float32),
        ],
        compiler_params=pltpu.CompilerParams(use_tc_tiling_on_sc=False),
    )
    def k(dst_hbm, ones_hbm, z_hbm, out_hbm, dst_v, ones_v, acc):
        cid = lax.axis_index("c")
        sid = lax.axis_index("s")
        wid = sid * _NC + cid
        pltpu.sync_copy(dst_hbm.at[pl.ds(wid * cpw, cpw)], dst_v)
        pltpu.sync_copy(ones_hbm, ones_v)
        pltpu.sync_copy(z_hbm, acc.at[pl.ds(sid * rpt, rpt)])
        plsc.subcore_barrier()

        def body(j, carry):
            pltpu.sync_copy(ones_v, acc.at[dst_v.at[j]], add=True)
            return carry

        lax.fori_loop(0, cpw, body, 0)
        plsc.subcore_barrier()
        pltpu.sync_copy(acc.at[pl.ds(sid * rpt, rpt)],
                        out_hbm.at[cid, pl.ds(sid * rpt, rpt)])

    return k(dst2d, ones_h, zrows)


def _edge_segsum(g, src2d, dst2d, zrows):
    D = g.shape[1]
    n_rows = src2d.shape[0]
    cpw = n_rows // (_NC * _NS)
    rpt = zrows.shape[0]
    n_acc = _NS * rpt
    mesh = plsc.VectorSubcoreMesh(core_axis_name="c", subcore_axis_name="s")

    K = 4
    ngroups = cpw // K

    @functools.partial(
        pl.kernel,
        out_type=jax.ShapeDtypeStruct((_NC, n_acc, D), jnp.float32),
        mesh=mesh,
        scratch_types=[
            pltpu.VMEM((cpw, _CH), jnp.int32),
            pltpu.VMEM((cpw, _CH), jnp.int32),
            pltpu.VMEM((2, K, _CH, D), jnp.float32),
            pltpu.VMEM_SHARED((n_acc, D), jnp.float32),
            pltpu.SemaphoreType.DMA((2,)),
            pltpu.SemaphoreType.DMA((2,)),
        ],
        compiler_params=pltpu.CompilerParams(use_tc_tiling_on_sc=False),
    )
    def k(g_hbm, src_hbm, dst_hbm, z_hbm, out_hbm,
          src_v, dst_v, rows_v, acc, gsem, ssem):
        cid = lax.axis_index("c")
        sid = lax.axis_index("s")
        wid = sid * _NC + cid
        pltpu.sync_copy(src_hbm.at[pl.ds(wid * cpw, cpw)], src_v)
        pltpu.sync_copy(dst_hbm.at[pl.ds(wid * cpw, cpw)], dst_v)
        pltpu.sync_copy(z_hbm, acc.at[pl.ds(sid * rpt, rpt)])
        plsc.subcore_barrier()

        def fire_gathers(i, p):
            for b in range(K):
                pltpu.async_copy(g_hbm.at[src_v.at[i * K + b]],
                                 rows_v.at[p, b], gsem.at[p])

        def drain_gathers(i, p):
            for b in range(K):
                pltpu.make_async_copy(g_hbm.at[src_v.at[i * K + b]],
                                      rows_v.at[p, b], gsem.at[p]).wait()

        def fire_scatters(i, p):
            for b in range(K):
                pltpu.async_copy(rows_v.at[p, b],
                                 acc.at[dst_v.at[i * K + b]], ssem.at[p],
                                 add=True)

        def drain_scatters(i, p):
            for b in range(K):
                pltpu.make_async_copy(rows_v.at[p, b],
                                      acc.at[dst_v.at[i * K + b]],
                                      ssem.at[p]).wait()

        fire_gathers(0, 0)

        def body(i2, carry):
            g0 = 2 * i2
            g1 = 2 * i2 + 1
            drain_gathers(g0, 0)
            fire_gathers(g1, 1)
            fire_scatters(g0, 0)
            drain_scatters(g0, 0)
            drain_gathers(g1, 1)

            @pl.when(g1 + 1 < ngroups)
            def _():
                fire_gathers(g1 + 1, 0)

            fire_scatters(g1, 1)
            drain_scatters(g1, 1)
            return carry

        lax.fori_loop(0, ngroups // 2, body, 0)
        plsc.subcore_barrier()
        pltpu.sync_copy(acc.at[pl.ds(sid * rpt, rpt)],
                        out_hbm.at[cid, pl.ds(sid * rpt, rpt)])

    return k(g, src2d, dst2d, zrows)


def _tc_matmul(x, W):
    N, F = x.shape
    H = W.shape[1]

    def body(x_ref, w_ref, o_ref):
        o_ref[...] = jnp.dot(x_ref[...], w_ref[...],
                             preferred_element_type=jnp.float32)

    return pl.pallas_call(
        body,
        grid=(N // _BR,),
        in_specs=[pl.BlockSpec((_BR, F), lambda i: (i, 0)),
                  pl.BlockSpec((F, H), lambda i: (0, 0))],
        out_specs=pl.BlockSpec((_BR, H), lambda i: (i, 0)),
        out_shape=jax.ShapeDtypeStruct((N, H), jnp.float32),
    )(x, W)


def _tc_scale(h, degP):
    N, H = h.shape

    def body(h_ref, d0_ref, d1_ref, g_ref, dinv_ref):
        deg = d0_ref[0] + d1_ref[0] + 1.0
        dinv = lax.rsqrt(deg)
        dinv_ref[...] = dinv
        g_ref[...] = h_ref[...] * dinv

    return pl.pallas_call(
        body,
        grid=(N // _BR,),
        in_specs=[pl.BlockSpec((_BR, H), lambda i: (i, 0)),
                  pl.BlockSpec((1, _BR, 1), lambda i: (0, i, 0)),
                  pl.BlockSpec((1, _BR, 1), lambda i: (1, i, 0))],
        out_specs=[pl.BlockSpec((_BR, H), lambda i: (i, 0)),
                   pl.BlockSpec((_BR, 1), lambda i: (i, 0))],
        out_shape=[jax.ShapeDtypeStruct((N, H), jnp.float32),
                   jax.ShapeDtypeStruct((N, 1), jnp.float32)],
    )(h, degP, degP)


def _tc_mid(g1, sp1, dinv, b1, W2):
    N, H = g1.shape
    C = W2.shape[1]

    def body(g1_ref, s0_ref, s1_ref, dinv_ref, b1_ref, w2_ref, g2_ref):
        s = s0_ref[0] + s1_ref[0] + g1_ref[...]
        t = jnp.maximum(s * dinv_ref[...] + b1_ref[...], 0.0)
        g2_ref[...] = jnp.dot(t, w2_ref[...],
                              preferred_element_type=jnp.float32) * dinv_ref[...]

    return pl.pallas_call(
        body,
        grid=(N // _BR,),
        in_specs=[pl.BlockSpec((_BR, H), lambda i: (i, 0)),
                  pl.BlockSpec((1, _BR, H), lambda i: (0, i, 0)),
                  pl.BlockSpec((1, _BR, H), lambda i: (1, i, 0)),
                  pl.BlockSpec((_BR, 1), lambda i: (i, 0)),
                  pl.BlockSpec((1, H), lambda i: (0, 0)),
                  pl.BlockSpec((H, C), lambda i: (0, 0))],
        out_specs=pl.BlockSpec((_BR, C), lambda i: (i, 0)),
        out_shape=jax.ShapeDtypeStruct((N, C), jnp.float32),
    )(g1, sp1, sp1, dinv, b1, W2)


def _tc_out(g2, sp2, dinv, b2):
    N, C = g2.shape

    def body(g2_ref, s0_ref, s1_ref, dinv_ref, b2_ref, o_ref):
        s = s0_ref[0] + s1_ref[0] + g2_ref[...]
        o_ref[...] = jax.nn.sigmoid(s * dinv_ref[...] + b2_ref[...])

    return pl.pallas_call(
        body,
        grid=(N // _BR,),
        in_specs=[pl.BlockSpec((_BR, C), lambda i: (i, 0)),
                  pl.BlockSpec((1, _BR, C), lambda i: (0, i, 0)),
                  pl.BlockSpec((1, _BR, C), lambda i: (1, i, 0)),
                  pl.BlockSpec((_BR, 1), lambda i: (i, 0)),
                  pl.BlockSpec((1, C), lambda i: (0, 0))],
        out_specs=pl.BlockSpec((_BR, C), lambda i: (i, 0)),
        out_shape=jax.ShapeDtypeStruct((N, C), jnp.float32),
    )(g2, sp2, sp2, dinv, b2)


def kernel(x, edge_index, W1, b1, W2, b2):
    N, F = x.shape
    H = W1.shape[1]
    C = W2.shape[1]
    E = edge_index.shape[1]

    src = edge_index[0].astype(jnp.int32)
    dst = edge_index[1].astype(jnp.int32)

    NW = _NC * _NS
    cpw = -(-E // (_CH * NW))
    cpw = -(-cpw // 8) * 8
    Epad = NW * cpw * _CH
    src2 = jnp.concatenate(
        [src, jnp.zeros((Epad - E,), jnp.int32)]).reshape(NW * cpw, _CH)
    dst2 = jnp.concatenate(
        [dst, jnp.full((Epad - E,), N, jnp.int32)]).reshape(NW * cpw, _CH)

    rpt = -(-(N + 1) // _NS)
    rpt = -(-rpt // 8) * 8
    zH = jnp.zeros((rpt, H), jnp.float32)
    zC = jnp.zeros((rpt, C), jnp.float32)
    z1 = jnp.zeros((rpt, 1), jnp.float32)
    ones = jnp.ones((_CH, 1), jnp.float32)

    h1 = _tc_matmul(x, W1)
    degP = _edge_deg(dst2, ones, z1)
    g1, dinv = _tc_scale(h1, degP)
    sp1 = _edge_segsum(g1, src2, dst2, zH)
    g2 = _tc_mid(g1, sp1, dinv, b1.reshape(1, H), W2)
    sp2 = _edge_segsum(g2, src2, dst2, zC)
    return _tc_out(g2, sp2, dinv, b2.reshape(1, C))
